# baseline (speedup 1.0000x reference)
import functools

import jax
import jax.numpy as jnp
from jax import lax
from jax.experimental import pallas as pl
from jax.experimental.pallas import tpu as pltpu
from jax.experimental.pallas import tpu_sc as plsc

BATCH = 4096
FIELD_COUNT = 26
DIM = 16
NNZ = 212992
N_SEG = BATCH * FIELD_COUNT

NUM_CORES = 2
NUM_SUBCORES = 16
NUM_LANES = 16

NSLOT = 4
BLK = 128
NB_TOT = NNZ // BLK
NBMAX = -(-NB_TOT // NUM_SUBCORES)

HALF_SEG = N_SEG // NUM_CORES
ROWS_PER_SUB = HALF_SEG // NUM_SUBCORES

_mesh = plsc.VectorSubcoreMesh(core_axis_name="c", subcore_axis_name="s")

_cp = pltpu.CompilerParams(
    needs_layout_passes=False, use_tc_tiling_on_sc=False
)


@functools.partial(
    pl.kernel,
    out_type=jax.ShapeDtypeStruct((N_SEG, DIM), jnp.float32),
    mesh=_mesh,
    scratch_types=[
        pltpu.VMEM((16,), jnp.int32),
        pltpu.VMEM((NBMAX, BLK), jnp.int32),
        pltpu.VMEM((NBMAX, BLK), jnp.float32),
        pltpu.VMEM((NBMAX, BLK), jnp.int32),
        pltpu.VMEM((NSLOT, BLK, DIM), jnp.float32),
        pltpu.VMEM((NSLOT, BLK), jnp.int32),
        pltpu.VMEM_SHARED((HALF_SEG, DIM), jnp.float32),
        pltpu.SemaphoreType.DMA((NSLOT,)),
        pltpu.SemaphoreType.DMA((NSLOT,)),
    ],
    compiler_params=_cp,
)
def _embed_sum(ids_hbm, w_hbm, segs_hbm, emb_hbm, split_hbm, out_hbm,
               split_v, ids_v, w_v, segs_v, rows_v, segidx_v, acc_sh,
               g_sem, s_sem):
    c = lax.axis_index("c")
    s = lax.axis_index("s")

    pltpu.sync_copy(split_hbm, split_v)
    split = split_v[...][1]
    hi0 = -(-(-(-split // BLK)) // 8) * 8
    lo1 = (split // BLK) // 8 * 8
    lo_b = jnp.where(c == 0, 0, lo1)
    hi_b = jnp.where(c == 0, hi0, NB_TOT)
    nb8 = (hi_b - lo_b) // 8
    my_lo = lo_b + 8 * ((nb8 * s) // NUM_SUBCORES)
    my_hi = lo_b + 8 * ((nb8 * (s + 1)) // NUM_SUBCORES)
    n_my = my_hi - my_lo
    base = jnp.minimum(my_lo, NB_TOT - NBMAX)
    base = pl.multiple_of(base, 8)
    ofs = my_lo - base

    pltpu.sync_copy(ids_hbm.at[pl.ds(base, NBMAX)], ids_v)
    pltpu.sync_copy(w_hbm.at[pl.ds(base, NBMAX)], w_v)
    pltpu.sync_copy(segs_hbm.at[pl.ds(base, NBMAX)], segs_v)

    seg_base = c * HALF_SEG

    def g_desc(jj, slot):
        return pltpu.make_async_copy(
            emb_hbm.at[ids_v.at[ofs + jj]], rows_v.at[slot], g_sem.at[slot])

    def s_desc(slot):
        return pltpu.make_async_copy(
            rows_v.at[slot], acc_sh.at[segidx_v.at[slot]], s_sem.at[slot])

    def compute(jj, slot):
        for g in range(BLK // NUM_LANES):
            sl = pl.ds(g * NUM_LANES, NUM_LANES)
            segv = segs_v[ofs + jj, sl]
            wv = w_v[ofs + jj, sl]
            own = (segv >= seg_base) & (segv < seg_base + HALF_SEG)
            wok = jnp.where(own, wv, 0.0)
            segloc = jnp.where(own, segv - seg_base, 0)
            segidx_v[slot, sl] = segloc
            ridx = lax.iota(jnp.int32, NUM_LANES) + (g * NUM_LANES)
            rslot = rows_v.at[slot]
            for d in range(DIM):
                cidx = jnp.full((NUM_LANES,), d, jnp.int32)
                v = plsc.load_gather(rslot, [ridx, cidx])
                plsc.store_scatter(rslot, [ridx, cidx], v * wok)

    for i in range(2):
        pl.when(i < n_my)(lambda i=i: g_desc(i, i).start())

    zero = jnp.zeros((NUM_LANES,), jnp.float32)
    zbuf = rows_v.at[NSLOT - 1]

    @pl.loop(0, BLK)
    def _(i):
        zbuf[i, :] = zero

    row0 = s * ROWS_PER_SUB

    @pl.loop(0, ROWS_PER_SUB // BLK)
    def _(k):
        pltpu.sync_copy(zbuf, acc_sh.at[pl.ds(row0 + k * BLK, BLK)])

    plsc.subcore_barrier()

    @pl.loop(0, (n_my + NSLOT - 1) // NSLOT)
    def _(k):
        for i in range(NSLOT):
            jj = k * NSLOT + i
            b2 = (i + 2) % NSLOT

            @pl.when(jj + 2 < n_my)
            def _():
                pl.when(jj >= 2)(lambda: s_desc(b2).wait())
                g_desc(jj + 2, b2).start()

            @pl.when(jj < n_my)
            def _():
                g_desc(jj, i).wait()
                compute(jj, i)
                s_desc(i).start(add=True)

    for i in range(NSLOT):
        pl.when((n_my >= NSLOT) | (i < n_my))(lambda i=i: s_desc(i).wait())

    plsc.subcore_barrier()

    pltpu.sync_copy(acc_sh.at[pl.ds(row0, ROWS_PER_SUB)],
                    out_hbm.at[pl.ds(seg_base + row0, ROWS_PER_SUB)])


def kernel(feat_ids, feat_weights, segment_ids, embedding):
    segs32 = segment_ids.astype(jnp.int32)
    ids = feat_ids.astype(jnp.int32).reshape(NB_TOT, BLK)
    w = feat_weights.astype(jnp.float32).reshape(NB_TOT, BLK)
    segs = segs32.reshape(NB_TOT, BLK)

    split_vec = jnp.searchsorted(
        segs32, jnp.arange(16, dtype=jnp.int32) * HALF_SEG
    ).astype(jnp.int32)

    pooled = _embed_sum(ids, w, segs, embedding, split_vec)
    return pooled.reshape(BATCH, FIELD_COUNT * DIM)


# speedup vs baseline: 1.1042x; 1.1042x over previous
import functools

import jax
import jax.numpy as jnp
from jax import lax
from jax.experimental import pallas as pl
from jax.experimental.pallas import tpu as pltpu
from jax.experimental.pallas import tpu_sc as plsc

BATCH = 4096
FIELD_COUNT = 26
DIM = 16
NNZ = 212992
N_SEG = BATCH * FIELD_COUNT

NUM_CORES = 2
NUM_SUBCORES = 16
NUM_LANES = 16

NSLOT = 4
BLK = 128
NB_TOT = NNZ // BLK
NBMAX = -(-NB_TOT // NUM_SUBCORES)

HALF_SEG = N_SEG // NUM_CORES
ROWS_PER_SUB = HALF_SEG // NUM_SUBCORES

_mesh = plsc.VectorSubcoreMesh(core_axis_name="c", subcore_axis_name="s")

_cp = pltpu.CompilerParams(
    needs_layout_passes=False, use_tc_tiling_on_sc=False
)


@functools.partial(
    pl.kernel,
    out_type=jax.ShapeDtypeStruct((N_SEG, DIM), jnp.float32),
    mesh=_mesh,
    scratch_types=[
        pltpu.VMEM((16,), jnp.int32),
        pltpu.VMEM((NBMAX, BLK), jnp.int32),
        pltpu.VMEM((NBMAX, BLK), jnp.float32),
        pltpu.VMEM((NBMAX, BLK), jnp.int32),
        pltpu.VMEM((NSLOT, BLK, DIM), jnp.float32),
        pltpu.VMEM((NSLOT, BLK), jnp.int32),
        pltpu.VMEM_SHARED((HALF_SEG, DIM), jnp.float32),
        pltpu.SemaphoreType.DMA((NSLOT,)),
        pltpu.SemaphoreType.DMA((NSLOT,)),
        pltpu.SemaphoreType.DMA,
    ],
    compiler_params=_cp,
)
def _embed_sum(ids_hbm, w_hbm, segs_hbm, emb_hbm, split_hbm, out_hbm,
               split_v, ids_v, w_v, segs_v, rows_v, segidx_v, acc_sh,
               g_sem, s_sem, z_sem):
    c = lax.axis_index("c")
    s = lax.axis_index("s")

    pltpu.sync_copy(split_hbm, split_v)
    split = split_v[...][1]
    hi0 = -(-(-(-split // BLK)) // 8) * 8
    lo1 = (split // BLK) // 8 * 8
    lo_b = jnp.where(c == 0, 0, lo1)
    hi_b = jnp.where(c == 0, hi0, NB_TOT)
    nb8 = (hi_b - lo_b) // 8
    my_lo = lo_b + 8 * ((nb8 * s) // NUM_SUBCORES)
    my_hi = lo_b + 8 * ((nb8 * (s + 1)) // NUM_SUBCORES)
    n_my = my_hi - my_lo
    base = jnp.minimum(my_lo, NB_TOT - NBMAX)
    base = pl.multiple_of(base, 8)
    ofs = my_lo - base

    pltpu.sync_copy(ids_hbm.at[pl.ds(base, NBMAX)], ids_v)
    pltpu.sync_copy(w_hbm.at[pl.ds(base, NBMAX)], w_v)
    pltpu.sync_copy(segs_hbm.at[pl.ds(base, NBMAX)], segs_v)

    seg_base = c * HALF_SEG

    def g_desc(jj, slot):
        return pltpu.make_async_copy(
            emb_hbm.at[ids_v.at[ofs + jj]], rows_v.at[slot], g_sem.at[slot])

    def s_desc(slot):
        return pltpu.make_async_copy(
            rows_v.at[slot], acc_sh.at[segidx_v.at[slot]], s_sem.at[slot])

    def compute(jj, slot):
        for g in range(BLK // NUM_LANES):
            sl = pl.ds(g * NUM_LANES, NUM_LANES)
            segv = segs_v[ofs + jj, sl]
            wv = w_v[ofs + jj, sl]
            own = (segv >= seg_base) & (segv < seg_base + HALF_SEG)
            wok = jnp.where(own, wv, 0.0)
            segloc = jnp.where(own, segv - seg_base, 0)
            segidx_v[slot, sl] = segloc
            for i in range(NUM_LANES):
                wb = jnp.take(wok, jnp.full((NUM_LANES,), i, jnp.int32))
                r = g * NUM_LANES + i
                rows_v[slot, r, :] = rows_v[slot, r, :] * wb

    for i in range(2):
        pl.when(i < n_my)(lambda i=i: g_desc(i, i).start())

    zero = jnp.zeros((NUM_LANES,), jnp.float32)
    zbuf = rows_v.at[NSLOT - 1]

    @pl.loop(0, BLK)
    def _(i):
        zbuf[i, :] = zero

    row0 = s * ROWS_PER_SUB

    def z_desc(k):
        return pltpu.make_async_copy(
            zbuf, acc_sh.at[pl.ds(row0 + k * BLK, BLK)], z_sem)

    @pl.loop(0, ROWS_PER_SUB // BLK)
    def _(k):
        z_desc(k).start()

    @pl.loop(0, ROWS_PER_SUB // BLK)
    def _(k):
        z_desc(k).wait()

    plsc.subcore_barrier()

    @pl.loop(0, (n_my + NSLOT - 1) // NSLOT)
    def _(k):
        for i in range(NSLOT):
            jj = k * NSLOT + i
            b2 = (i + 2) % NSLOT

            @pl.when(jj + 2 < n_my)
            def _():
                pl.when(jj >= 2)(lambda: s_desc(b2).wait())
                g_desc(jj + 2, b2).start()

            @pl.when(jj < n_my)
            def _():
                g_desc(jj, i).wait()
                compute(jj, i)
                s_desc(i).start(add=True)

    for i in range(NSLOT):
        pl.when((n_my >= NSLOT) | (i < n_my))(lambda i=i: s_desc(i).wait())

    plsc.subcore_barrier()

    pltpu.sync_copy(acc_sh.at[pl.ds(row0, ROWS_PER_SUB)],
                    out_hbm.at[pl.ds(seg_base + row0, ROWS_PER_SUB)])


def kernel(feat_ids, feat_weights, segment_ids, embedding):
    segs32 = segment_ids.astype(jnp.int32)
    ids = feat_ids.astype(jnp.int32).reshape(NB_TOT, BLK)
    w = feat_weights.astype(jnp.float32).reshape(NB_TOT, BLK)
    segs = segs32.reshape(NB_TOT, BLK)

    split_vec = jnp.searchsorted(
        segs32, jnp.arange(16, dtype=jnp.int32) * HALF_SEG
    ).astype(jnp.int32)

    pooled = _embed_sum(ids, w, segs, embedding, split_vec)
    return pooled.reshape(BATCH, FIELD_COUNT * DIM)
